# baseline (speedup 1.0000x reference)
import functools

import jax
import jax.numpy as jnp
from jax import lax
from jax.experimental import pallas as pl
from jax.experimental.pallas import tpu as pltpu
from jax.experimental.pallas import tpu_sc as plsc

N = 10000
E = 320000
NP = 10240
RPS = NP // 16
NC = 2
NS = 16
W = NC * NS
EPW = E // W
B = 80
NB = EPW // B

_MESH = plsc.VectorSubcoreMesh(
    core_axis_name="c", subcore_axis_name="s", num_cores=NC, num_subcores=NS
)
_SC_PARAMS = pltpu.CompilerParams(use_tc_tiling_on_sc=False)


def _worker_id():
    c = lax.axis_index("c")
    s = lax.axis_index("s")
    return c * NS + s, c, s


def _zero_rows(zbuf, nrows, f):
    def body(i, _):
        for h in range(f // 16):
            zbuf[i, pl.ds(h * 16, 16)] = jnp.zeros((16,), jnp.float32)
        return 0

    lax.fori_loop(0, nrows, body, 0)


def _zero_flat(zbuf, nelem):
    def body(i, _):
        zbuf[pl.ds(i * 16, 16)] = jnp.zeros((16,), jnp.float32)
        return 0

    lax.fori_loop(0, nelem // 16, body, 0)


@functools.partial(
    pl.kernel,
    out_type=jax.ShapeDtypeStruct((NC, NP), jnp.float32),
    mesh=_MESH,
    scratch_types=[
        pltpu.VMEM((EPW,), jnp.int32),
        pltpu.VMEM((128,), jnp.float32),
        pltpu.VMEM((RPS,), jnp.float32),
        pltpu.VMEM_SHARED((NP,), jnp.float32),
        pltpu.SemaphoreType.DMA,
    ],
    compiler_params=_SC_PARAMS,
)
def _sc_degree(dst_hbm, out_hbm, dst_v, ones_v, zbuf, hist_sh, sem):
    wid, c, s = _worker_id()
    _zero_flat(zbuf, RPS)

    def setone(i, _):
        ones_v[pl.ds(i * 16, 16)] = jnp.ones((16,), jnp.float32)
        return 0

    lax.fori_loop(0, (B // 16) + 1, setone, 0)
    pltpu.sync_copy(zbuf, hist_sh.at[pl.ds(s * RPS, RPS)])
    pltpu.sync_copy(dst_hbm.at[pl.ds(wid * EPW, EPW)], dst_v)
    plsc.subcore_barrier()

    def fire(j, _):
        pltpu.async_copy(ones_v.at[pl.ds(0, B)],
                         hist_sh.at[dst_v.at[pl.ds(j * B, B)]],
                         sem, add=True)
        return 0

    lax.fori_loop(0, NB, fire, 0)

    def drain(j, _):
        pltpu.make_async_copy(ones_v.at[pl.ds(0, B)],
                              hist_sh.at[dst_v.at[pl.ds(0, B)]],
                              sem).wait()
        return 0

    lax.fori_loop(0, NB, drain, 0)

    plsc.subcore_barrier()
    pltpu.sync_copy(hist_sh.at[pl.ds(s * RPS, RPS)],
                    out_hbm.at[c, pl.ds(s * RPS, RPS)])


def _make_sc_aggregate(f, chunk):
    nchunks = NB // chunk
    assert NB % chunk == 0

    @functools.partial(
        pl.kernel,
        out_type=jax.ShapeDtypeStruct((NC, NP, f), jnp.float32),
        mesh=_MESH,
        scratch_types=[
            pltpu.VMEM((EPW,), jnp.int32),
            pltpu.VMEM((EPW,), jnp.int32),
            pltpu.VMEM((2, chunk, B, f), jnp.float32),
            pltpu.VMEM((RPS, f), jnp.float32),
            pltpu.VMEM_SHARED((NP, f), jnp.float32),
            pltpu.SemaphoreType.DMA,
            pltpu.SemaphoreType.DMA,
        ],
        compiler_params=_SC_PARAMS,
    )
    def agg(table_hbm, src_hbm, dst_hbm, out_hbm,
            src_v, dst_v, rows_v, zbuf, acc_sh, gsem, ssem):
        wid, c, s = _worker_id()
        _zero_rows(zbuf, RPS, f)
        pltpu.sync_copy(zbuf, acc_sh.at[pl.ds(s * RPS, RPS)])
        pltpu.sync_copy(src_hbm.at[pl.ds(wid * EPW, EPW)], src_v)
        pltpu.sync_copy(dst_hbm.at[pl.ds(wid * EPW, EPW)], dst_v)
        plsc.subcore_barrier()

        def gfire(t, buf):
            def body(j, _):
                pltpu.async_copy(
                    table_hbm.at[src_v.at[pl.ds((t * chunk + j) * B, B)]],
                    rows_v.at[buf, j], gsem)
                return 0

            lax.fori_loop(0, chunk, body, 0)

        def gdrain():
            def body(j, _):
                pltpu.make_async_copy(table_hbm.at[src_v.at[pl.ds(0, B)]],
                                      rows_v.at[0, 0], gsem).wait()
                return 0

            lax.fori_loop(0, chunk, body, 0)

        def sfire(t, buf):
            def body(j, _):
                pltpu.async_copy(
                    rows_v.at[buf, j],
                    acc_sh.at[dst_v.at[pl.ds((t * chunk + j) * B, B)]],
                    ssem, add=True)
                return 0

            lax.fori_loop(0, chunk, body, 0)

        def sdrain():
            def body(j, _):
                pltpu.make_async_copy(rows_v.at[0, 0],
                                      acc_sh.at[dst_v.at[pl.ds(0, B)]],
                                      ssem).wait()
                return 0

            lax.fori_loop(0, chunk, body, 0)

        gfire(0, 0)
        for t in range(nchunks):
            gdrain()
            if t + 1 < nchunks:
                gfire(t + 1, (t + 1) % 2)
            sfire(t, t % 2)
            sdrain()

        plsc.subcore_barrier()
        pltpu.sync_copy(acc_sh.at[pl.ds(s * RPS, RPS)],
                        out_hbm.at[c, pl.ds(s * RPS, RPS)])

    return agg


_sc_aggregate16 = _make_sc_aggregate(16, 5)
_sc_aggregate32 = _make_sc_aggregate(32, 5)


def _tc_scale(cnt, x, w1):
    def body(cnt_ref, x_ref, w_ref, d_ref, u_ref):
        xw = jnp.dot(x_ref[...], w_ref[...],
                     preferred_element_type=jnp.float32)
        deg = cnt_ref[0, :N] + cnt_ref[1, :N] + 1.0
        d = lax.rsqrt(deg)[:, None]
        d_ref[...] = d
        u_ref[...] = d * xw

    return pl.pallas_call(
        body,
        out_shape=[
            jax.ShapeDtypeStruct((N, 1), jnp.float32),
            jax.ShapeDtypeStruct((N, 16), jnp.float32),
        ],
    )(cnt, x, w1)


def _tc_layer2(acc1, u, d, b1, wcat):
    def body(a_ref, u_ref, d_ref, b_ref, w_ref, v_ref):
        d = d_ref[...]
        t = d * (a_ref[0, :N] + a_ref[1, :N] + u_ref[...]) + b_ref[...]
        h = jnp.maximum(t, 0.0)
        v_ref[...] = d * jnp.dot(h, w_ref[...],
                                 preferred_element_type=jnp.float32)

    return pl.pallas_call(
        body,
        out_shape=jax.ShapeDtypeStruct((N, 32), jnp.float32),
    )(acc1, u, d, b1, wcat)


def _tc_final(acc2, v, d, bmu, bls):
    def body(a_ref, v_ref, d_ref, bmu_ref, bls_ref, mu_ref, ls_ref):
        o = d_ref[...] * (a_ref[0, :N] + a_ref[1, :N] + v_ref[...])
        mu_ref[...] = o[:, :16] + bmu_ref[...]
        ls_ref[...] = o[:, 16:] + bls_ref[...]

    return pl.pallas_call(
        body,
        out_shape=[
            jax.ShapeDtypeStruct((N, 16), jnp.float32),
            jax.ShapeDtypeStruct((N, 16), jnp.float32),
        ],
    )(acc2, v, d, bmu, bls)


@jax.jit
def kernel(x, edge_index, W1, b1, W_mu, b_mu, W_ls, b_ls):
    src = edge_index[0].astype(jnp.int32)
    dst = edge_index[1].astype(jnp.int32)

    cnt = _sc_degree(dst)
    d, u = _tc_scale(cnt, x, W1)

    acc1 = _sc_aggregate16(u, src, dst)
    wcat = jnp.concatenate([W_mu, W_ls], axis=1)
    v = _tc_layer2(acc1, u, d, b1.reshape(1, 16), wcat)

    acc2 = _sc_aggregate32(v, src, dst)
    return _tc_final(acc2, v, d, b_mu.reshape(1, 16), b_ls.reshape(1, 16))


# speedup vs baseline: 1.0302x; 1.0302x over previous
import functools

import jax
import jax.numpy as jnp
from jax import lax
from jax.experimental import pallas as pl
from jax.experimental.pallas import tpu as pltpu
from jax.experimental.pallas import tpu_sc as plsc

N = 10000
E = 320000
NP = 10240
RPS = NP // 16
NC = 2
NS = 16
W = NC * NS
EPW = E // W
B = 125
NB = EPW // B

_MESH = plsc.VectorSubcoreMesh(
    core_axis_name="c", subcore_axis_name="s", num_cores=NC, num_subcores=NS
)
_SC_PARAMS = pltpu.CompilerParams(use_tc_tiling_on_sc=False)


def _worker_id():
    c = lax.axis_index("c")
    s = lax.axis_index("s")
    return c * NS + s, c, s


def _zero_rows(zbuf, nrows, f):
    def body(i, _):
        for h in range(f // 16):
            zbuf[i, pl.ds(h * 16, 16)] = jnp.zeros((16,), jnp.float32)
        return 0

    lax.fori_loop(0, nrows, body, 0)


def _zero_flat(zbuf, nelem):
    def body(i, _):
        zbuf[pl.ds(i * 16, 16)] = jnp.zeros((16,), jnp.float32)
        return 0

    lax.fori_loop(0, nelem // 16, body, 0)


@functools.partial(
    pl.kernel,
    out_type=jax.ShapeDtypeStruct((NC, NP), jnp.float32),
    mesh=_MESH,
    scratch_types=[
        pltpu.VMEM((NB, B), jnp.int32),
        pltpu.VMEM((128,), jnp.float32),
        pltpu.VMEM((RPS,), jnp.float32),
        pltpu.VMEM_SHARED((NP,), jnp.float32),
        pltpu.SemaphoreType.DMA,
    ],
    compiler_params=_SC_PARAMS,
)
def _sc_degree(dst_hbm, out_hbm, dst_v, ones_v, zbuf, hist_sh, sem):
    wid, c, s = _worker_id()
    _zero_flat(zbuf, RPS)

    def setone(i, _):
        ones_v[pl.ds(i * 16, 16)] = jnp.ones((16,), jnp.float32)
        return 0

    lax.fori_loop(0, (B // 16) + 1, setone, 0)
    pltpu.sync_copy(zbuf, hist_sh.at[pl.ds(s * RPS, RPS)])
    pltpu.sync_copy(dst_hbm.at[wid], dst_v)
    plsc.subcore_barrier()

    def fire(j, _):
        pltpu.async_copy(ones_v.at[pl.ds(0, B)], hist_sh.at[dst_v.at[j]],
                         sem, add=True)
        return 0

    lax.fori_loop(0, NB, fire, 0)

    def drain(j, _):
        pltpu.make_async_copy(ones_v.at[pl.ds(0, B)], hist_sh.at[dst_v.at[0]],
                              sem).wait()
        return 0

    lax.fori_loop(0, NB, drain, 0)

    plsc.subcore_barrier()
    pltpu.sync_copy(hist_sh.at[pl.ds(s * RPS, RPS)],
                    out_hbm.at[c, pl.ds(s * RPS, RPS)])


def _make_sc_aggregate(f, chunk):
    nchunks = NB // chunk
    assert NB % chunk == 0

    @functools.partial(
        pl.kernel,
        out_type=jax.ShapeDtypeStruct((NC, NP, f), jnp.float32),
        mesh=_MESH,
        scratch_types=[
            pltpu.VMEM((NB, B), jnp.int32),
            pltpu.VMEM((NB, B), jnp.int32),
            pltpu.VMEM((2, chunk, B, f), jnp.float32),
            pltpu.VMEM((RPS, f), jnp.float32),
            pltpu.VMEM_SHARED((NP, f), jnp.float32),
            pltpu.SemaphoreType.DMA,
            pltpu.SemaphoreType.DMA,
        ],
        compiler_params=_SC_PARAMS,
    )
    def agg(table_hbm, src_hbm, dst_hbm, out_hbm,
            src_v, dst_v, rows_v, zbuf, acc_sh, gsem, ssem):
        wid, c, s = _worker_id()
        _zero_rows(zbuf, RPS, f)
        pltpu.sync_copy(zbuf, acc_sh.at[pl.ds(s * RPS, RPS)])
        pltpu.sync_copy(src_hbm.at[wid], src_v)
        pltpu.sync_copy(dst_hbm.at[wid], dst_v)
        plsc.subcore_barrier()

        def gfire(t, buf):
            def body(j, _):
                pltpu.async_copy(table_hbm.at[src_v.at[t * chunk + j]],
                                 rows_v.at[buf, j], gsem)
                return 0

            lax.fori_loop(0, chunk, body, 0)

        def gdrain():
            def body(j, _):
                pltpu.make_async_copy(table_hbm.at[src_v.at[0]],
                                      rows_v.at[0, 0], gsem).wait()
                return 0

            lax.fori_loop(0, chunk, body, 0)

        def sfire(t, buf):
            def body(j, _):
                pltpu.async_copy(rows_v.at[buf, j],
                                 acc_sh.at[dst_v.at[t * chunk + j]],
                                 ssem, add=True)
                return 0

            lax.fori_loop(0, chunk, body, 0)

        def sdrain():
            def body(j, _):
                pltpu.make_async_copy(rows_v.at[0, 0],
                                      acc_sh.at[dst_v.at[0]], ssem).wait()
                return 0

            lax.fori_loop(0, chunk, body, 0)

        gfire(0, 0)
        for t in range(nchunks):
            gdrain()
            if t + 1 < nchunks:
                gfire(t + 1, (t + 1) % 2)
            sfire(t, t % 2)
            sdrain()

        plsc.subcore_barrier()
        pltpu.sync_copy(acc_sh.at[pl.ds(s * RPS, RPS)],
                        out_hbm.at[c, pl.ds(s * RPS, RPS)])

    return agg


_sc_aggregate16 = _make_sc_aggregate(16, 16)
_sc_aggregate32 = _make_sc_aggregate(32, 8)


def _tc_scale(cnt, x, w1):
    def body(cnt_ref, x_ref, w_ref, d_ref, u_ref):
        xw = jnp.dot(x_ref[...], w_ref[...],
                     preferred_element_type=jnp.float32)
        deg = cnt_ref[0, :N] + cnt_ref[1, :N] + 1.0
        d = lax.rsqrt(deg)[:, None]
        d_ref[...] = d
        u_ref[...] = d * xw

    return pl.pallas_call(
        body,
        out_shape=[
            jax.ShapeDtypeStruct((N, 1), jnp.float32),
            jax.ShapeDtypeStruct((N, 16), jnp.float32),
        ],
    )(cnt, x, w1)


def _tc_layer2(acc1, u, d, b1, wcat):
    def body(a_ref, u_ref, d_ref, b_ref, w_ref, v_ref):
        d = d_ref[...]
        t = d * (a_ref[0, :N] + a_ref[1, :N] + u_ref[...]) + b_ref[...]
        h = jnp.maximum(t, 0.0)
        v_ref[...] = d * jnp.dot(h, w_ref[...],
                                 preferred_element_type=jnp.float32)

    return pl.pallas_call(
        body,
        out_shape=jax.ShapeDtypeStruct((N, 32), jnp.float32),
    )(acc1, u, d, b1, wcat)


def _tc_final(acc2, v, d, bmu, bls):
    def body(a_ref, v_ref, d_ref, bmu_ref, bls_ref, mu_ref, ls_ref):
        o = d_ref[...] * (a_ref[0, :N] + a_ref[1, :N] + v_ref[...])
        mu_ref[...] = o[:, :16] + bmu_ref[...]
        ls_ref[...] = o[:, 16:] + bls_ref[...]

    return pl.pallas_call(
        body,
        out_shape=[
            jax.ShapeDtypeStruct((N, 16), jnp.float32),
            jax.ShapeDtypeStruct((N, 16), jnp.float32),
        ],
    )(acc2, v, d, bmu, bls)


@jax.jit
def kernel(x, edge_index, W1, b1, W_mu, b_mu, W_ls, b_ls):
    dst = edge_index[1].astype(jnp.int32).reshape(W, NB, B)
    src = lax.optimization_barrier(edge_index)[0].astype(jnp.int32)
    src = src.reshape(W, NB, B)

    cnt = _sc_degree(dst)
    d, u = _tc_scale(cnt, x, W1)

    acc1 = _sc_aggregate16(u, src, dst)
    wcat = jnp.concatenate([W_mu, W_ls], axis=1)
    v = _tc_layer2(acc1, u, d, b1.reshape(1, 16), wcat)

    acc2 = _sc_aggregate32(v, src, dst)
    return _tc_final(acc2, v, d, b_mu.reshape(1, 16), b_ls.reshape(1, 16))


# speedup vs baseline: 1.0726x; 1.0411x over previous
import functools

import jax
import jax.numpy as jnp
from jax import lax
from jax.experimental import pallas as pl
from jax.experimental.pallas import tpu as pltpu
from jax.experimental.pallas import tpu_sc as plsc

N = 10000
E = 320000
NP = 10240
RPS = NP // 16
NC = 2
NS = 16
W = NC * NS
EPW = E // W
B = 125
NB = EPW // B

_MESH = plsc.VectorSubcoreMesh(
    core_axis_name="c", subcore_axis_name="s", num_cores=NC, num_subcores=NS
)
_SC_PARAMS = pltpu.CompilerParams(use_tc_tiling_on_sc=False)


def _worker_id():
    c = lax.axis_index("c")
    s = lax.axis_index("s")
    return c * NS + s, c, s


def _zero_rows(zbuf, nrows, f):
    def body(i, _):
        for h in range(f // 16):
            zbuf[i, pl.ds(h * 16, 16)] = jnp.zeros((16,), jnp.float32)
        return 0

    lax.fori_loop(0, nrows, body, 0)


def _zero_flat(zbuf, nelem):
    def body(i, _):
        zbuf[pl.ds(i * 16, 16)] = jnp.zeros((16,), jnp.float32)
        return 0

    lax.fori_loop(0, nelem // 16, body, 0)


@functools.partial(
    pl.kernel,
    out_type=jax.ShapeDtypeStruct((NC, NP), jnp.float32),
    mesh=_MESH,
    scratch_types=[
        pltpu.VMEM((NB, B), jnp.int32),
        pltpu.VMEM((128,), jnp.float32),
        pltpu.VMEM((RPS,), jnp.float32),
        pltpu.VMEM_SHARED((NP,), jnp.float32),
        pltpu.SemaphoreType.DMA,
    ],
    compiler_params=_SC_PARAMS,
)
def _sc_degree(dst_hbm, out_hbm, dst_v, ones_v, zbuf, hist_sh, sem):
    wid, c, s = _worker_id()
    _zero_flat(zbuf, RPS)

    def setone(i, _):
        ones_v[pl.ds(i * 16, 16)] = jnp.ones((16,), jnp.float32)
        return 0

    lax.fori_loop(0, (B // 16) + 1, setone, 0)
    pltpu.sync_copy(zbuf, hist_sh.at[pl.ds(s * RPS, RPS)])
    pltpu.sync_copy(dst_hbm.at[wid], dst_v)
    plsc.subcore_barrier()

    def fire(j, _):
        pltpu.async_copy(ones_v.at[pl.ds(0, B)], hist_sh.at[dst_v.at[j]],
                         sem, add=True)
        return 0

    lax.fori_loop(0, NB, fire, 0)

    def drain(j, _):
        pltpu.make_async_copy(ones_v.at[pl.ds(0, B)], hist_sh.at[dst_v.at[0]],
                              sem).wait()
        return 0

    lax.fori_loop(0, NB, drain, 0)

    plsc.subcore_barrier()
    pltpu.sync_copy(hist_sh.at[pl.ds(s * RPS, RPS)],
                    out_hbm.at[c, pl.ds(s * RPS, RPS)])


def _make_sc_aggregate(f, chunk):
    nchunks = NB // chunk
    assert NB % chunk == 0

    @functools.partial(
        pl.kernel,
        out_type=jax.ShapeDtypeStruct((NC, NP, f), jnp.float32),
        mesh=_MESH,
        scratch_types=[
            pltpu.VMEM((NB, B), jnp.int32),
            pltpu.VMEM((NB, B), jnp.int32),
            pltpu.VMEM((2, chunk, B, f), jnp.float32),
            pltpu.VMEM((RPS, f), jnp.float32),
            pltpu.VMEM_SHARED((NP, f), jnp.float32),
            pltpu.SemaphoreType.DMA,
            pltpu.SemaphoreType.DMA,
        ],
        compiler_params=_SC_PARAMS,
    )
    def agg(table_hbm, src_hbm, dst_hbm, out_hbm,
            src_v, dst_v, rows_v, zbuf, acc_sh, gsem, ssem):
        wid, c, s = _worker_id()
        _zero_rows(zbuf, RPS, f)
        pltpu.sync_copy(zbuf, acc_sh.at[pl.ds(s * RPS, RPS)])
        pltpu.sync_copy(src_hbm.at[wid], src_v)
        pltpu.sync_copy(dst_hbm.at[wid], dst_v)
        plsc.subcore_barrier()

        def gfire(t, buf):
            def body(j, _):
                pltpu.async_copy(table_hbm.at[src_v.at[t * chunk + j]],
                                 rows_v.at[buf, j], gsem)
                return 0

            lax.fori_loop(0, chunk, body, 0)

        def gdrain():
            def body(j, _):
                pltpu.make_async_copy(table_hbm.at[src_v.at[0]],
                                      rows_v.at[0, 0], gsem).wait()
                return 0

            lax.fori_loop(0, chunk, body, 0)

        def sfire(t, buf):
            def body(j, _):
                pltpu.async_copy(rows_v.at[buf, j],
                                 acc_sh.at[dst_v.at[t * chunk + j]],
                                 ssem, add=True)
                return 0

            lax.fori_loop(0, chunk, body, 0)

        def sdrain():
            def body(j, _):
                pltpu.make_async_copy(rows_v.at[0, 0],
                                      acc_sh.at[dst_v.at[0]], ssem).wait()
                return 0

            lax.fori_loop(0, chunk, body, 0)

        gfire(0, 0)
        for t in range(nchunks):
            gdrain()
            if t + 1 < nchunks:
                gfire(t + 1, (t + 1) % 2)
            sfire(t, t % 2)
            sdrain()

        plsc.subcore_barrier()
        pltpu.sync_copy(acc_sh.at[pl.ds(s * RPS, RPS)],
                        out_hbm.at[c, pl.ds(s * RPS, RPS)])

    return agg


_sc_aggregate16 = _make_sc_aggregate(16, 16)
_sc_aggregate32 = _make_sc_aggregate(32, 8)


def _tc_matmul1(x, w1):
    def body(x_ref, w_ref, o_ref):
        o_ref[...] = jnp.dot(x_ref[...], w_ref[...],
                             preferred_element_type=jnp.float32)

    return pl.pallas_call(
        body,
        out_shape=jax.ShapeDtypeStruct((N, 16), jnp.float32),
    )(x, w1)


def _tc_scale(cnt, xw):
    def body(cnt_ref, xw_ref, d_ref, u_ref):
        deg = cnt_ref[0, :N] + cnt_ref[1, :N] + 1.0
        d = lax.rsqrt(deg)[:, None]
        d_ref[...] = d
        u_ref[...] = d * xw_ref[...]

    return pl.pallas_call(
        body,
        out_shape=[
            jax.ShapeDtypeStruct((N, 1), jnp.float32),
            jax.ShapeDtypeStruct((N, 16), jnp.float32),
        ],
    )(cnt, xw)


def _tc_layer2(acc1, u, d, b1, wcat):
    def body(a_ref, u_ref, d_ref, b_ref, w_ref, v_ref):
        d = d_ref[...]
        t = d * (a_ref[0, :N] + a_ref[1, :N] + u_ref[...]) + b_ref[...]
        h = jnp.maximum(t, 0.0)
        v_ref[...] = d * jnp.dot(h, w_ref[...],
                                 preferred_element_type=jnp.float32)

    return pl.pallas_call(
        body,
        out_shape=jax.ShapeDtypeStruct((N, 32), jnp.float32),
    )(acc1, u, d, b1, wcat)


def _tc_final(acc2, v, d, bmu, bls):
    def body(a_ref, v_ref, d_ref, bmu_ref, bls_ref, mu_ref, ls_ref):
        o = d_ref[...] * (a_ref[0, :N] + a_ref[1, :N] + v_ref[...])
        mu_ref[...] = o[:, :16] + bmu_ref[...]
        ls_ref[...] = o[:, 16:] + bls_ref[...]

    return pl.pallas_call(
        body,
        out_shape=[
            jax.ShapeDtypeStruct((N, 16), jnp.float32),
            jax.ShapeDtypeStruct((N, 16), jnp.float32),
        ],
    )(acc2, v, d, bmu, bls)


@jax.jit
def kernel(x, edge_index, W1, b1, W_mu, b_mu, W_ls, b_ls):
    src = edge_index[0].astype(jnp.int32).reshape(W, NB, B)
    dst = edge_index[1].astype(jnp.int32).reshape(W, NB, B)

    cnt = _sc_degree(dst)
    xw = _tc_matmul1(x, W1)
    d, u = _tc_scale(cnt, xw)

    acc1 = _sc_aggregate16(u, src, dst)
    wcat = jnp.concatenate([W_mu, W_ls], axis=1)
    v = _tc_layer2(acc1, u, d, b1.reshape(1, 16), wcat)

    acc2 = _sc_aggregate32(v, src, dst)
    return _tc_final(acc2, v, d, b_mu.reshape(1, 16), b_ls.reshape(1, 16))


# speedup vs baseline: 1.1039x; 1.0292x over previous
import functools

import jax
import jax.numpy as jnp
from jax import lax
from jax.experimental import pallas as pl
from jax.experimental.pallas import tpu as pltpu
from jax.experimental.pallas import tpu_sc as plsc

N = 10000
E = 320000
NP = 10240
RPS = NP // 16
NC = 2
NS = 16
W = NC * NS
EPW = E // W
B = 125
NB = EPW // B

_MESH = plsc.VectorSubcoreMesh(
    core_axis_name="c", subcore_axis_name="s", num_cores=NC, num_subcores=NS
)
_SC_PARAMS = pltpu.CompilerParams(use_tc_tiling_on_sc=False)


def _worker_id():
    c = lax.axis_index("c")
    s = lax.axis_index("s")
    return c * NS + s, c, s


def _zero_rows(zbuf, nrows, f):
    def body(i, _):
        for h in range(f // 16):
            zbuf[i, pl.ds(h * 16, 16)] = jnp.zeros((16,), jnp.float32)
        return 0

    lax.fori_loop(0, nrows, body, 0)


def _zero_flat(zbuf, nelem):
    def body(i, _):
        zbuf[pl.ds(i * 16, 16)] = jnp.zeros((16,), jnp.float32)
        return 0

    lax.fori_loop(0, nelem // 16, body, 0)


@functools.partial(
    pl.kernel,
    out_type=jax.ShapeDtypeStruct((NC, NP), jnp.float32),
    mesh=_MESH,
    scratch_types=[
        pltpu.VMEM((NB, B), jnp.int32),
        pltpu.VMEM((128,), jnp.float32),
        pltpu.VMEM((RPS,), jnp.float32),
        pltpu.VMEM_SHARED((NP,), jnp.float32),
        pltpu.SemaphoreType.DMA,
    ],
    compiler_params=_SC_PARAMS,
)
def _sc_degree(dst_hbm, out_hbm, dst_v, ones_v, zbuf, hist_sh, sem):
    wid, c, s = _worker_id()
    _zero_flat(zbuf, RPS)

    def setone(i, _):
        ones_v[pl.ds(i * 16, 16)] = jnp.ones((16,), jnp.float32)
        return 0

    lax.fori_loop(0, (B // 16) + 1, setone, 0)
    pltpu.sync_copy(zbuf, hist_sh.at[pl.ds(s * RPS, RPS)])
    pltpu.sync_copy(dst_hbm.at[wid], dst_v)
    plsc.subcore_barrier()

    def fire(j, _):
        pltpu.async_copy(ones_v.at[pl.ds(0, B)], hist_sh.at[dst_v.at[j]],
                         sem, add=True)
        return 0

    lax.fori_loop(0, NB, fire, 0)

    def drain(j, _):
        pltpu.make_async_copy(ones_v.at[pl.ds(0, B)], hist_sh.at[dst_v.at[0]],
                              sem).wait()
        return 0

    lax.fori_loop(0, NB, drain, 0)

    plsc.subcore_barrier()
    pltpu.sync_copy(hist_sh.at[pl.ds(s * RPS, RPS)],
                    out_hbm.at[c, pl.ds(s * RPS, RPS)])


def _make_sc_aggregate(f, chunk):
    nchunks = NB // chunk
    assert NB % chunk == 0

    @functools.partial(
        pl.kernel,
        out_type=jax.ShapeDtypeStruct((NC, NP, f), jnp.float32),
        mesh=_MESH,
        scratch_types=[
            pltpu.VMEM((NB, B), jnp.int32),
            pltpu.VMEM((NB, B), jnp.int32),
            pltpu.VMEM((2, chunk, B, f), jnp.float32),
            pltpu.VMEM((RPS // 8, f), jnp.float32),
            pltpu.VMEM_SHARED((NP, f), jnp.float32),
            pltpu.SemaphoreType.DMA,
            pltpu.SemaphoreType.DMA,
        ],
        compiler_params=_SC_PARAMS,
    )
    def agg(table_hbm, src_hbm, dst_hbm, out_hbm,
            src_v, dst_v, rows_v, zbuf, acc_sh, gsem, ssem):
        wid, c, s = _worker_id()
        _zero_rows(zbuf, RPS // 8, f)
        for z in range(8):
            pltpu.sync_copy(
                zbuf, acc_sh.at[pl.ds(s * RPS + z * (RPS // 8), RPS // 8)])
        pltpu.sync_copy(src_hbm.at[wid], src_v)
        pltpu.sync_copy(dst_hbm.at[wid], dst_v)
        plsc.subcore_barrier()

        def gfire(t, buf):
            def body(j, _):
                pltpu.async_copy(table_hbm.at[src_v.at[t * chunk + j]],
                                 rows_v.at[buf, j], gsem)
                return 0

            lax.fori_loop(0, chunk, body, 0)

        def gdrain():
            def body(j, _):
                pltpu.make_async_copy(table_hbm.at[src_v.at[0]],
                                      rows_v.at[0, 0], gsem).wait()
                return 0

            lax.fori_loop(0, chunk, body, 0)

        def sfire(t, buf):
            def body(j, _):
                pltpu.async_copy(rows_v.at[buf, j],
                                 acc_sh.at[dst_v.at[t * chunk + j]],
                                 ssem, add=True)
                return 0

            lax.fori_loop(0, chunk, body, 0)

        def sdrain():
            def body(j, _):
                pltpu.make_async_copy(rows_v.at[0, 0],
                                      acc_sh.at[dst_v.at[0]], ssem).wait()
                return 0

            lax.fori_loop(0, chunk, body, 0)

        gfire(0, 0)
        for t in range(nchunks):
            gdrain()
            if t + 1 < nchunks:
                gfire(t + 1, (t + 1) % 2)
            sfire(t, t % 2)
            sdrain()

        plsc.subcore_barrier()
        pltpu.sync_copy(acc_sh.at[pl.ds(s * RPS, RPS)],
                        out_hbm.at[c, pl.ds(s * RPS, RPS)])

    return agg


_sc_aggregate16 = _make_sc_aggregate(16, 20)
_sc_aggregate32 = _make_sc_aggregate(32, 10)


def _tc_matmul1(x, w1):
    def body(x_ref, w_ref, o_ref):
        o_ref[...] = jnp.dot(x_ref[...], w_ref[...],
                             preferred_element_type=jnp.float32)

    return pl.pallas_call(
        body,
        out_shape=jax.ShapeDtypeStruct((N, 16), jnp.float32),
    )(x, w1)


def _tc_scale(cnt, xw):
    def body(cnt_ref, xw_ref, d_ref, u_ref):
        deg = cnt_ref[0, :N] + cnt_ref[1, :N] + 1.0
        d = lax.rsqrt(deg)[:, None]
        d_ref[...] = d
        u_ref[...] = d * xw_ref[...]

    return pl.pallas_call(
        body,
        out_shape=[
            jax.ShapeDtypeStruct((N, 1), jnp.float32),
            jax.ShapeDtypeStruct((N, 16), jnp.float32),
        ],
    )(cnt, xw)


def _tc_layer2(acc1, u, d, b1, wcat):
    def body(a_ref, u_ref, d_ref, b_ref, w_ref, v_ref):
        d = d_ref[...]
        t = d * (a_ref[0, :N] + a_ref[1, :N] + u_ref[...]) + b_ref[...]
        h = jnp.maximum(t, 0.0)
        v_ref[...] = d * jnp.dot(h, w_ref[...],
                                 preferred_element_type=jnp.float32)

    return pl.pallas_call(
        body,
        out_shape=jax.ShapeDtypeStruct((N, 32), jnp.float32),
    )(acc1, u, d, b1, wcat)


def _tc_final(acc2, v, d, bmu, bls):
    def body(a_ref, v_ref, d_ref, bmu_ref, bls_ref, mu_ref, ls_ref):
        o = d_ref[...] * (a_ref[0, :N] + a_ref[1, :N] + v_ref[...])
        mu_ref[...] = o[:, :16] + bmu_ref[...]
        ls_ref[...] = o[:, 16:] + bls_ref[...]

    return pl.pallas_call(
        body,
        out_shape=[
            jax.ShapeDtypeStruct((N, 16), jnp.float32),
            jax.ShapeDtypeStruct((N, 16), jnp.float32),
        ],
    )(acc2, v, d, bmu, bls)


@jax.jit
def kernel(x, edge_index, W1, b1, W_mu, b_mu, W_ls, b_ls):
    src = edge_index[0].astype(jnp.int32).reshape(W, NB, B)
    dst = edge_index[1].astype(jnp.int32).reshape(W, NB, B)

    cnt = _sc_degree(dst)
    xw = _tc_matmul1(x, W1)
    d, u = _tc_scale(cnt, xw)

    acc1 = _sc_aggregate16(u, src, dst)
    wcat = jnp.concatenate([W_mu, W_ls], axis=1)
    v = _tc_layer2(acc1, u, d, b1.reshape(1, 16), wcat)

    acc2 = _sc_aggregate32(v, src, dst)
    return _tc_final(acc2, v, d, b_mu.reshape(1, 16), b_ls.reshape(1, 16))
